# baseline (speedup 1.0000x reference)
import jax
import jax.numpy as jnp
from jax import lax
from jax.experimental import pallas as pl
from jax.experimental.pallas import tpu as pltpu
from jax.experimental.pallas import tpu_sc as plsc

_VOCAB = 100000
_EMBED = 64
_B = 16384
_C = 20
_NEG = 100
_LE_LAMBDA = 1e-08
_NPAIR = 16
_CP = 32
_NEGP = 112

_NW = 32
_SPW = _B // _NW
_S = 8
_NCHUNK = _SPW // _S


def _sc_dots_body(in_embed, out16, il, plf, nlf, pidx,
                  pos_dot_hbm, neg_dot_hbm, pair_rows_hbm,
                  *scr):
    idx_in = scr[0:3]
    idx_pos = scr[3:6]
    idx_neg = scr[6:9]
    in_rows = scr[9:12]
    pos_rows = scr[12:15]
    neg_rows = scr[15:18]
    pos_dot = scr[18:21]
    neg_dot = scr[21:24]
    pair_idx_v, pair_rows_v, gsem, isem, osem = scr[24:29]

    cid = lax.axis_index("c")
    sid = lax.axis_index("s")
    wid = sid * 2 + cid

    @pl.when(wid == 0)
    def _():
        pltpu.sync_copy(pidx, pair_idx_v)
        pltpu.async_copy(in_embed.at[pair_idx_v], pair_rows_v, gsem).wait()
        pltpu.sync_copy(pair_rows_v, pair_rows_hbm)

    def issue_idx(g, b):
        base = wid * _SPW + g * _S
        pltpu.async_copy(il.at[pl.ds(base, _S)], idx_in[b], isem)
        pltpu.async_copy(plf.at[pl.ds(base * _C, _S * _C)], idx_pos[b], isem)
        pltpu.async_copy(nlf.at[pl.ds(base * _NEG, _S * _NEG)], idx_neg[b], isem)

    def wait_idx(b):
        pltpu.make_async_copy(il.at[pl.ds(0, _S)], idx_in[b], isem).wait()
        pltpu.make_async_copy(plf.at[pl.ds(0, _S * _C)], idx_pos[b], isem).wait()
        pltpu.make_async_copy(nlf.at[pl.ds(0, _S * _NEG)], idx_neg[b], isem).wait()

    def issue_gathers(b):
        pltpu.async_copy(in_embed.at[idx_in[b]], in_rows[b], gsem)
        pltpu.async_copy(out16.at[idx_pos[b].at[pl.ds(0, 128)]],
                         pos_rows[b].at[pl.ds(0, 128)], gsem)
        pltpu.async_copy(out16.at[idx_pos[b].at[pl.ds(128, 32)]],
                         pos_rows[b].at[pl.ds(128, 32)], gsem)
        for j in range(6):
            pltpu.async_copy(out16.at[idx_neg[b].at[pl.ds(j * 128, 128)]],
                             neg_rows[b].at[pl.ds(j * 128, 128)], gsem)
        pltpu.async_copy(out16.at[idx_neg[b].at[pl.ds(768, 32)]],
                         neg_rows[b].at[pl.ds(768, 32)], gsem)

    def wait_gathers(b):
        pltpu.make_async_copy(in_embed.at[idx_in[b]], in_rows[b], gsem).wait()
        pltpu.make_async_copy(out16.at[idx_pos[b]], pos_rows[b], gsem).wait()
        pltpu.make_async_copy(out16.at[idx_neg[b]], neg_rows[b], gsem).wait()

    def issue_out(g, b):
        base = wid * _SPW + g * _S
        pltpu.async_copy(pos_dot[b], pos_dot_hbm.at[pl.ds(base, _S)], osem)
        pltpu.async_copy(neg_dot[b], neg_dot_hbm.at[pl.ds(base, _S)], osem)

    def wait_out(b):
        pltpu.make_async_copy(pos_dot[b], pos_dot_hbm.at[pl.ds(0, _S)], osem).wait()
        pltpu.make_async_copy(neg_dot[b], neg_dot_hbm.at[pl.ds(0, _S)], osem).wait()

    lane = lax.iota(jnp.int32, 16)
    perms = {k: lane ^ k for k in (1, 2, 4, 8)}
    kmasks = {k: (lane & k) == 0 for k in (1, 2, 4, 8)}
    pe = (2 * lane) & 15
    po = pe | 1
    lo8 = lane < 8
    HIMASK = jnp.int32(-65536)

    def compute(b):
        def bfly(a, k):
            return a + jnp.take_along_axis(a, perms[k], axis=0)

        def merge(a, bv, k):
            return jnp.where(kmasks[k], bfly(a, k), bfly(bv, k))

        def sample_body(i, carry2):
            u = [in_rows[b][i, pl.ds(16 * j, 16)] for j in range(4)]
            ud = []
            for h in range(2):
                ue = jnp.where(lo8, jnp.take_along_axis(u[2 * h], pe, axis=0),
                               jnp.take_along_axis(u[2 * h + 1], pe, axis=0))
                uo = jnp.where(lo8, jnp.take_along_axis(u[2 * h], po, axis=0),
                               jnp.take_along_axis(u[2 * h + 1], po, axis=0))
                ud += [ue, uo]
            und = [-x for x in ud]

            def dot_partial(rows_v, r, uv):
                b0 = plsc.bitcast(rows_v[r, pl.ds(0, 32)], jnp.int32)
                b1 = plsc.bitcast(rows_v[r, pl.ds(32, 32)], jnp.int32)
                lo0 = plsc.bitcast(b0 << 16, jnp.float32)
                hi0 = plsc.bitcast(b0 & HIMASK, jnp.float32)
                lo1 = plsc.bitcast(b1 << 16, jnp.float32)
                hi1 = plsc.bitcast(b1 & HIMASK, jnp.float32)
                return ((lo0 * uv[0] + hi0 * uv[1])
                        + (lo1 * uv[2] + hi1 * uv[3]))

            def group16(rows_v, r0, uv, count):
                level = [dot_partial(rows_v, r0 + j, uv) for j in range(count)]
                for k in (1, 2, 4, 8):
                    if len(level) == 1:
                        level = [bfly(level[0], k)]
                    else:
                        level = [merge(level[2 * m], level[2 * m + 1], k)
                                 for m in range(len(level) // 2)]
                return level[0]

            pos_dot[b][i, pl.ds(0, 16)] = group16(pos_rows[b], i * _C, ud, 16)
            pos_dot[b][i, pl.ds(16, 16)] = group16(pos_rows[b], i * _C + 16, ud, 4)
            for g16 in range(6):
                neg_dot[b][i, pl.ds(g16 * 16, 16)] = group16(
                    neg_rows[b], i * _NEG + g16 * 16, und, 16)
            neg_dot[b][i, pl.ds(96, 16)] = group16(
                neg_rows[b], i * _NEG + 96, und, 4)
            return carry2

        lax.fori_loop(0, _S, sample_body, None)

    issue_idx(0, 0)
    issue_idx(1, 1)
    issue_idx(2, 2)
    wait_idx(0)
    issue_gathers(0)
    wait_idx(1)
    issue_gathers(1)

    def step_body(step, carry):
        for b in (0, 1, 2):
            g = 3 * step + b
            s2 = (b + 2) % 3

            @pl.when(g + 2 <= _NCHUNK - 1)
            def _():
                wait_idx(s2)
                issue_gathers(s2)

            @pl.when(g <= _NCHUNK - 1)
            def _():
                wait_gathers(b)

                @pl.when(g + 3 <= _NCHUNK - 1)
                def _():
                    issue_idx(g + 3, b)

                @pl.when(g >= 3)
                def _():
                    wait_out(b)

                compute(b)
                issue_out(g, b)
        return carry

    lax.fori_loop(0, (_NCHUNK + 2) // 3, step_body, None)

    wait_out(0)
    wait_out(1)
    wait_out(2)


_sc_dots = pl.kernel(
    _sc_dots_body,
    out_type=[
        jax.ShapeDtypeStruct((_B, _CP), jnp.float32),
        jax.ShapeDtypeStruct((_B, _NEGP), jnp.float32),
        jax.ShapeDtypeStruct((2 * _NPAIR, _EMBED), jnp.float32),
    ],
    mesh=plsc.VectorSubcoreMesh(core_axis_name="c", subcore_axis_name="s"),
    compiler_params=pltpu.CompilerParams(
        use_tc_tiling_on_sc=False, needs_layout_passes=False),
    scratch_types=(
        [pltpu.VMEM((_S,), jnp.int32)] * 3
        + [pltpu.VMEM((_S * _C,), jnp.int32)] * 3
        + [pltpu.VMEM((_S * _NEG,), jnp.int32)] * 3
        + [pltpu.VMEM((_S, _EMBED), jnp.float32)] * 3
        + [pltpu.VMEM((_S * _C, _EMBED), jnp.bfloat16)] * 3
        + [pltpu.VMEM((_S * _NEG, _EMBED), jnp.bfloat16)] * 3
        + [pltpu.VMEM((_S, _CP), jnp.float32)] * 3
        + [pltpu.VMEM((_S, _NEGP), jnp.float32)] * 3
        + [
            pltpu.VMEM((2 * _NPAIR,), jnp.int32),
            pltpu.VMEM((2 * _NPAIR, _EMBED), jnp.float32),
            pltpu.SemaphoreType.DMA,
            pltpu.SemaphoreType.DMA,
            pltpu.SemaphoreType.DMA,
        ]
    ),
)


def _log_sigmoid(x):
    return jnp.minimum(x, 0.0) - jnp.log1p(jnp.exp(-jnp.abs(x)))


def _tc_reduce_body(pos_ref, neg_ref, pr_ref, loss_ref, hier_ref):
    i = pl.program_id(0)
    pmask = lax.broadcasted_iota(jnp.int32, (_TC_BLK, _CP), 1) < _C
    nmask = lax.broadcasted_iota(jnp.int32, (_TC_BLK, _NEGP), 1) < _NEG
    s = (jnp.sum(jnp.where(pmask, _log_sigmoid(pos_ref[...]), 0.0))
         + jnp.sum(jnp.where(nmask, _log_sigmoid(neg_ref[...]), 0.0)))
    part = -s / _B

    @pl.when(i == 0)
    def _():
        d = pr_ref[0:_NPAIR, :] - pr_ref[_NPAIR:2 * _NPAIR, :]
        h = 0.5 * _LE_LAMBDA * jnp.sum(d * d)
        hier_ref[0, 0] = h
        loss_ref[0, 0] = part + h

    @pl.when(i > 0)
    def _():
        loss_ref[0, 0] = loss_ref[0, 0] + part


_TC_BLK = 1024


def _tc_reduce(pos_dot, neg_dot, pair_rows):
    return pl.pallas_call(
        _tc_reduce_body,
        grid=(_B // _TC_BLK,),
        in_specs=[
            pl.BlockSpec((_TC_BLK, _CP), lambda i: (i, 0)),
            pl.BlockSpec((_TC_BLK, _NEGP), lambda i: (i, 0)),
            pl.BlockSpec((2 * _NPAIR, _EMBED), lambda i: (0, 0)),
        ],
        out_specs=[
            pl.BlockSpec((1, 1), lambda i: (0, 0), memory_space=pltpu.SMEM),
            pl.BlockSpec((1, 1), lambda i: (0, 0), memory_space=pltpu.SMEM),
        ],
        out_shape=[
            jax.ShapeDtypeStruct((1, 1), jnp.float32),
            jax.ShapeDtypeStruct((1, 1), jnp.float32),
        ],
    )(pos_dot, neg_dot, pair_rows)


def kernel(in_embed, out_embed, input_labels, pos_labels, neg_labels, pairs):
    il = input_labels.astype(jnp.int32)
    plf = pos_labels.reshape(-1).astype(jnp.int32)
    nlf = neg_labels.reshape(-1).astype(jnp.int32)
    pidx = jnp.concatenate([pairs[:, 0], pairs[:, 1]]).astype(jnp.int32)
    out16 = out_embed.astype(jnp.bfloat16)
    pos_dot, neg_dot, pair_rows = _sc_dots(
        in_embed, out16, il, plf, nlf, pidx)
    loss, hier = _tc_reduce(pos_dot, neg_dot, pair_rows)
    return (loss[0, 0], hier[0, 0])


# speedup vs baseline: 1.0793x; 1.0793x over previous
import jax
import jax.numpy as jnp
from jax import lax
from jax.experimental import pallas as pl
from jax.experimental.pallas import tpu as pltpu
from jax.experimental.pallas import tpu_sc as plsc

_VOCAB = 100000
_EMBED = 64
_B = 16384
_C = 20
_NEG = 100
_LE_LAMBDA = 1e-08
_NPAIR = 16
_CP = 32
_NEGP = 112

_NW = 32
_SPW = _B // _NW
_S = 8
_NCHUNK = _SPW // _S


def _sc_dots_body(in_embed, out16, il, plf, nlf, pidx,
                  pos_dot_hbm, neg_dot_hbm, pair_rows_hbm,
                  *scr):
    idx_in = scr[0:2]
    idx_pos = scr[2:4]
    idx_neg = scr[4:6]
    in_rows = scr[6:8]
    pos_rows = scr[8:10]
    neg_rows = scr[10:12]
    pos_dot = scr[12:14]
    neg_dot = scr[14:16]
    pair_idx_v, pair_rows_v, gsem, isem, osem = scr[16:21]

    cid = lax.axis_index("c")
    sid = lax.axis_index("s")
    wid = sid * 2 + cid

    @pl.when(wid == 0)
    def _():
        pltpu.sync_copy(pidx, pair_idx_v)
        pltpu.async_copy(in_embed.at[pair_idx_v], pair_rows_v, gsem).wait()
        pltpu.sync_copy(pair_rows_v, pair_rows_hbm)

    def issue_idx(g, b):
        base = wid * _SPW + g * _S
        pltpu.async_copy(il.at[pl.ds(base, _S)], idx_in[b], isem)
        pltpu.async_copy(plf.at[pl.ds(base * _C, _S * _C)], idx_pos[b], isem)
        pltpu.async_copy(nlf.at[pl.ds(base * _NEG, _S * _NEG)], idx_neg[b], isem)

    def wait_idx(b):
        pltpu.make_async_copy(il.at[pl.ds(0, _S)], idx_in[b], isem).wait()
        pltpu.make_async_copy(plf.at[pl.ds(0, _S * _C)], idx_pos[b], isem).wait()
        pltpu.make_async_copy(nlf.at[pl.ds(0, _S * _NEG)], idx_neg[b], isem).wait()

    def issue_gathers(b):
        pltpu.async_copy(in_embed.at[idx_in[b]], in_rows[b], gsem)
        pltpu.async_copy(out16.at[idx_pos[b].at[pl.ds(0, 128)]],
                         pos_rows[b].at[pl.ds(0, 128)], gsem)
        pltpu.async_copy(out16.at[idx_pos[b].at[pl.ds(128, 32)]],
                         pos_rows[b].at[pl.ds(128, 32)], gsem)
        for j in range(6):
            pltpu.async_copy(out16.at[idx_neg[b].at[pl.ds(j * 128, 128)]],
                             neg_rows[b].at[pl.ds(j * 128, 128)], gsem)
        pltpu.async_copy(out16.at[idx_neg[b].at[pl.ds(768, 32)]],
                         neg_rows[b].at[pl.ds(768, 32)], gsem)

    def wait_gathers(b):
        pltpu.make_async_copy(in_embed.at[idx_in[b]], in_rows[b], gsem).wait()
        pltpu.make_async_copy(out16.at[idx_pos[b]], pos_rows[b], gsem).wait()
        pltpu.make_async_copy(out16.at[idx_neg[b]], neg_rows[b], gsem).wait()

    def issue_out(g, b):
        base = wid * _SPW + g * _S
        pltpu.async_copy(pos_dot[b], pos_dot_hbm.at[pl.ds(base, _S)], osem)
        pltpu.async_copy(neg_dot[b], neg_dot_hbm.at[pl.ds(base, _S)], osem)

    def wait_out(b):
        pltpu.make_async_copy(pos_dot[b], pos_dot_hbm.at[pl.ds(0, _S)], osem).wait()
        pltpu.make_async_copy(neg_dot[b], neg_dot_hbm.at[pl.ds(0, _S)], osem).wait()

    lane = lax.iota(jnp.int32, 16)
    perms = {k: lane ^ k for k in (1, 2, 4, 8)}
    kmasks = {k: (lane & k) == 0 for k in (1, 2, 4, 8)}

    def compute(b):
        def bfly(a, k):
            return a + jnp.take_along_axis(a, perms[k], axis=0)

        def merge(a, bv, k):
            return jnp.where(kmasks[k], bfly(a, k), bfly(bv, k))

        def sample_body(i, carry2):
            ud = [in_rows[b][i, pl.ds(16 * j, 16)] for j in range(4)]
            und = [-x for x in ud]

            def dot_partial(rows_v, r, uv):
                acc = rows_v[r, pl.ds(0, 16)] * uv[0]
                for j in range(1, 4):
                    acc = acc + rows_v[r, pl.ds(16 * j, 16)] * uv[j]
                return acc

            def group16(rows_v, r0, uv, count):
                level = [dot_partial(rows_v, r0 + j, uv) for j in range(count)]
                for k in (1, 2, 4, 8):
                    if len(level) == 1:
                        level = [bfly(level[0], k)]
                    else:
                        level = [merge(level[2 * m], level[2 * m + 1], k)
                                 for m in range(len(level) // 2)]
                return level[0]

            pos_dot[b][i, pl.ds(0, 16)] = group16(pos_rows[b], i * _C, ud, 16)
            pos_dot[b][i, pl.ds(16, 16)] = group16(pos_rows[b], i * _C + 16, ud, 4)
            for g16 in range(6):
                neg_dot[b][i, pl.ds(g16 * 16, 16)] = group16(
                    neg_rows[b], i * _NEG + g16 * 16, und, 16)
            neg_dot[b][i, pl.ds(96, 16)] = group16(
                neg_rows[b], i * _NEG + 96, und, 4)
            return carry2

        lax.fori_loop(0, _S, sample_body, None)

    issue_idx(0, 0)
    wait_idx(0)
    issue_gathers(0)
    issue_idx(1, 1)

    def step_body(step, carry):
        for b in (0, 1):
            g = 2 * step + b
            g2 = jnp.minimum(g + 2, _NCHUNK - 1)
            wait_idx(1 - b)
            issue_gathers(1 - b)
            wait_gathers(b)
            issue_idx(g2, b)

            @pl.when(step >= 1)
            def _():
                wait_out(b)

            compute(b)
            issue_out(g, b)
        return carry

    lax.fori_loop(0, _NCHUNK // 2, step_body, None)

    wait_gathers(0)
    wait_idx(1)
    wait_out(0)
    wait_out(1)


_sc_dots = pl.kernel(
    _sc_dots_body,
    out_type=[
        jax.ShapeDtypeStruct((_B, _CP), jnp.float32),
        jax.ShapeDtypeStruct((_B, _NEGP), jnp.float32),
        jax.ShapeDtypeStruct((2 * _NPAIR, _EMBED), jnp.float32),
    ],
    mesh=plsc.VectorSubcoreMesh(core_axis_name="c", subcore_axis_name="s"),
    compiler_params=pltpu.CompilerParams(
        use_tc_tiling_on_sc=False, needs_layout_passes=False),
    scratch_types=(
        [pltpu.VMEM((_S,), jnp.int32)] * 2
        + [pltpu.VMEM((_S * _C,), jnp.int32)] * 2
        + [pltpu.VMEM((_S * _NEG,), jnp.int32)] * 2
        + [pltpu.VMEM((_S, _EMBED), jnp.float32)] * 2
        + [pltpu.VMEM((_S * _C, _EMBED), jnp.float32)] * 2
        + [pltpu.VMEM((_S * _NEG, _EMBED), jnp.float32)] * 2
        + [pltpu.VMEM((_S, _CP), jnp.float32)] * 2
        + [pltpu.VMEM((_S, _NEGP), jnp.float32)] * 2
        + [
            pltpu.VMEM((2 * _NPAIR,), jnp.int32),
            pltpu.VMEM((2 * _NPAIR, _EMBED), jnp.float32),
            pltpu.SemaphoreType.DMA,
            pltpu.SemaphoreType.DMA,
            pltpu.SemaphoreType.DMA,
        ]
    ),
)


def _log_sigmoid(x):
    return jnp.minimum(x, 0.0) - jnp.log1p(jnp.exp(-jnp.abs(x)))


def _tc_reduce_body(pos_ref, neg_ref, pr_ref, loss_ref, hier_ref):
    i = pl.program_id(0)
    pmask = lax.broadcasted_iota(jnp.int32, (_TC_BLK, _CP), 1) < _C
    nmask = lax.broadcasted_iota(jnp.int32, (_TC_BLK, _NEGP), 1) < _NEG
    s = (jnp.sum(jnp.where(pmask, _log_sigmoid(pos_ref[...]), 0.0))
         + jnp.sum(jnp.where(nmask, _log_sigmoid(neg_ref[...]), 0.0)))
    part = -s / _B

    @pl.when(i == 0)
    def _():
        d = pr_ref[0:_NPAIR, :] - pr_ref[_NPAIR:2 * _NPAIR, :]
        h = 0.5 * _LE_LAMBDA * jnp.sum(d * d)
        hier_ref[0, 0] = h
        loss_ref[0, 0] = part + h

    @pl.when(i > 0)
    def _():
        loss_ref[0, 0] = loss_ref[0, 0] + part


_TC_BLK = 1024


def _tc_reduce(pos_dot, neg_dot, pair_rows):
    return pl.pallas_call(
        _tc_reduce_body,
        grid=(_B // _TC_BLK,),
        in_specs=[
            pl.BlockSpec((_TC_BLK, _CP), lambda i: (i, 0)),
            pl.BlockSpec((_TC_BLK, _NEGP), lambda i: (i, 0)),
            pl.BlockSpec((2 * _NPAIR, _EMBED), lambda i: (0, 0)),
        ],
        out_specs=[
            pl.BlockSpec((1, 1), lambda i: (0, 0), memory_space=pltpu.SMEM),
            pl.BlockSpec((1, 1), lambda i: (0, 0), memory_space=pltpu.SMEM),
        ],
        out_shape=[
            jax.ShapeDtypeStruct((1, 1), jnp.float32),
            jax.ShapeDtypeStruct((1, 1), jnp.float32),
        ],
    )(pos_dot, neg_dot, pair_rows)


def kernel(in_embed, out_embed, input_labels, pos_labels, neg_labels, pairs):
    il = input_labels.astype(jnp.int32)
    plf = pos_labels.reshape(-1).astype(jnp.int32)
    nlf = neg_labels.reshape(-1).astype(jnp.int32)
    pidx = jnp.concatenate([pairs[:, 0], pairs[:, 1]]).astype(jnp.int32)
    pos_dot, neg_dot, pair_rows = _sc_dots(
        in_embed, out_embed, il, plf, nlf, pidx)
    loss, hier = _tc_reduce(pos_dot, neg_dot, pair_rows)
    return (loss[0, 0], hier[0, 0])
